# Initial kernel scaffold; baseline (speedup 1.0000x reference)
import jax
import jax.numpy as jnp
from jax.experimental import pallas as pl


def kernel(wpe):
    raise NotImplementedError("write your pallas kernel here")


# speedup vs baseline: 2.7257x; 2.7257x over previous
import jax
import jax.numpy as jnp
from jax.experimental import pallas as pl

_BLOCK_SIZE = 8192
_N_EMBD = 1024
_ROWS_PER_BLOCK = 512


def _copy_body(x_ref, o_ref):
    o_ref[...] = x_ref[...]


def kernel(wpe):
    out = pl.pallas_call(
        _copy_body,
        grid=(_BLOCK_SIZE // _ROWS_PER_BLOCK,),
        in_specs=[pl.BlockSpec((_ROWS_PER_BLOCK, _N_EMBD), lambda i: (i, 0))],
        out_specs=pl.BlockSpec((_ROWS_PER_BLOCK, _N_EMBD), lambda i: (i, 0)),
        out_shape=jax.ShapeDtypeStruct((_BLOCK_SIZE, _N_EMBD), jnp.float32),
    )(wpe)
    return out[None]
